# Initial kernel scaffold; baseline (speedup 1.0000x reference)
import jax
import jax.numpy as jnp
from jax.experimental import pallas as pl


def kernel(x, edge_index, W_l, b_l, W_r):
    raise NotImplementedError("write your pallas kernel here")


# speedup vs baseline: 7.0290x; 7.0290x over previous
import functools

import jax
import jax.numpy as jnp
from jax import lax
from jax.experimental import pallas as pl
from jax.experimental.pallas import tpu as pltpu
from jax.experimental.pallas import tpu_sc as plsc

N = 10000
E = 320000
D = 128
DA = 144

NC = 2
NS = 16
NW = NC * NS
EPW = E // NW
CB = 80
NCHUNK = EPW // CB
NP = 10240
RPT = NP // NS


def _sc_aggregate(xaug, src3, dst3, zeros):
  mesh = plsc.VectorSubcoreMesh(core_axis_name="c", subcore_axis_name="s")

  @functools.partial(
      pl.kernel,
      out_type=jax.ShapeDtypeStruct((NC, NP, DA), jnp.float32),
      mesh=mesh,
      compiler_params=pltpu.CompilerParams(use_tc_tiling_on_sc=False),
      scratch_types=[
          pltpu.VMEM((NCHUNK, CB), jnp.int32),
          pltpu.VMEM((NCHUNK, CB), jnp.int32),
          pltpu.VMEM((CB, DA), jnp.float32),
          pltpu.VMEM_SHARED((NP, DA), jnp.float32),
          pltpu.SemaphoreType.DMA,
      ],
  )
  def agg_kernel(x_hbm, src_hbm, dst_hbm, z_hbm, out_hbm,
                 src_v, dst_v, gbuf, acc_sh, sem):
    cid = lax.axis_index("c")
    sid = lax.axis_index("s")
    wid = cid * NS + sid

    pltpu.sync_copy(src_hbm.at[wid], src_v)
    pltpu.sync_copy(dst_hbm.at[wid], dst_v)

    pltpu.sync_copy(z_hbm.at[pl.ds(sid * RPT, RPT)],
                    acc_sh.at[pl.ds(sid * RPT, RPT)])
    plsc.subcore_barrier()

    def body(j, _):
      pltpu.async_copy(x_hbm.at[src_v.at[j]], gbuf, sem).wait()
      pltpu.sync_copy(gbuf, acc_sh.at[dst_v.at[j]], add=True)
      return _

    lax.fori_loop(0, NCHUNK, body, None)
    plsc.subcore_barrier()

    pltpu.sync_copy(acc_sh.at[pl.ds(sid * RPT, RPT)],
                    out_hbm.at[cid, pl.ds(sid * RPT, RPT)])

  return agg_kernel(xaug, src3, dst3, zeros)


def _tc_finish_body(agg_ref, x_ref, wl_ref, bl_ref, wr_ref, out_ref):
  a = agg_ref[0] + agg_ref[1]
  deg = a[:, D:D + 1]
  mean = a[:, :D] / jnp.maximum(deg, 1.0)
  out = (
      lax.dot_general(mean, wl_ref[...], (((1,), (1,)), ((), ())),
                      preferred_element_type=jnp.float32)
      + lax.dot_general(x_ref[...], wr_ref[...], (((1,), (1,)), ((), ())),
                        preferred_element_type=jnp.float32)
      + bl_ref[...]
  )
  norm = jnp.sqrt(jnp.sum(out * out, axis=-1, keepdims=True))
  out_ref[...] = out / jnp.maximum(norm, 1e-12)


def _tc_finish(agg2, x, W_l, b_l2, W_r):
  blk = 2000
  grid = N // blk
  return pl.pallas_call(
      _tc_finish_body,
      grid=(grid,),
      in_specs=[
          pl.BlockSpec((NC, blk, DA), lambda i: (0, i, 0)),
          pl.BlockSpec((blk, D), lambda i: (i, 0)),
          pl.BlockSpec((D, D), lambda i: (0, 0)),
          pl.BlockSpec((1, D), lambda i: (0, 0)),
          pl.BlockSpec((D, D), lambda i: (0, 0)),
      ],
      out_specs=pl.BlockSpec((blk, D), lambda i: (i, 0)),
      out_shape=jax.ShapeDtypeStruct((N, D), jnp.float32),
  )(agg2, x, W_l, b_l2, W_r)


@jax.jit
def kernel(x, edge_index, W_l, b_l, W_r):
  xaug = jnp.concatenate(
      [x, jnp.ones((N, 1), jnp.float32), jnp.zeros((N, DA - D - 1), jnp.float32)],
      axis=1)
  src3 = edge_index[0].reshape(NW, NCHUNK, CB)
  dst3 = edge_index[1].reshape(NW, NCHUNK, CB)
  zeros = jnp.zeros((NP, DA), jnp.float32)
  agg2 = _sc_aggregate(xaug, src3, dst3, zeros)
  return _tc_finish(agg2, x, W_l, b_l.reshape(1, D), W_r)


# speedup vs baseline: 8.2402x; 1.1723x over previous
import functools

import jax
import jax.numpy as jnp
from jax import lax
from jax.experimental import pallas as pl
from jax.experimental.pallas import tpu as pltpu
from jax.experimental.pallas import tpu_sc as plsc

N = 10000
E = 320000
D = 128
DA = 144

NC = 2
NS = 16
NW = NC * NS
EPW = E // NW
CB = 50
NCHUNK = EPW // CB
NP = 10240
RPT = NP // NS


def _sc_aggregate(xaug, src3, dst3, zeros):
  mesh = plsc.VectorSubcoreMesh(core_axis_name="c", subcore_axis_name="s")

  @functools.partial(
      pl.kernel,
      out_type=jax.ShapeDtypeStruct((NC, NP, DA), jnp.float32),
      mesh=mesh,
      compiler_params=pltpu.CompilerParams(use_tc_tiling_on_sc=False),
      scratch_types=[
          pltpu.VMEM((NCHUNK, CB), jnp.int32),
          pltpu.VMEM((NCHUNK, CB), jnp.int32),
          pltpu.VMEM((CB, DA), jnp.float32),
          pltpu.VMEM((CB, DA), jnp.float32),
          pltpu.VMEM_SHARED((NP, DA), jnp.float32),
          pltpu.SemaphoreType.DMA,
          pltpu.SemaphoreType.DMA,
      ],
  )
  def agg_kernel(x_hbm, src_hbm, dst_hbm, z_hbm, out_hbm,
                 src_v, dst_v, gbufa, gbufb, acc_sh, sema, semb):
    cid = lax.axis_index("c")
    sid = lax.axis_index("s")
    wid = cid * NS + sid

    pltpu.sync_copy(src_hbm.at[wid], src_v)
    pltpu.sync_copy(dst_hbm.at[wid], dst_v)

    pltpu.sync_copy(z_hbm.at[pl.ds(sid * RPT, RPT)],
                    acc_sh.at[pl.ds(sid * RPT, RPT)])
    plsc.subcore_barrier()

    pltpu.async_copy(x_hbm.at[src_v.at[0]], gbufa, sema)

    @pl.loop(0, NCHUNK, step=2)
    def _(j):
      hb = pltpu.async_copy(x_hbm.at[src_v.at[j + 1]], gbufb, semb)
      pltpu.make_async_copy(x_hbm.at[pl.ds(0, CB)], gbufa, sema).wait()
      pltpu.sync_copy(gbufa, acc_sh.at[dst_v.at[j]], add=True)

      @pl.when(j + 2 < NCHUNK)
      def _():
        pltpu.async_copy(x_hbm.at[src_v.at[j + 2]], gbufa, sema)

      hb.wait()
      pltpu.sync_copy(gbufb, acc_sh.at[dst_v.at[j + 1]], add=True)

    plsc.subcore_barrier()

    pltpu.sync_copy(acc_sh.at[pl.ds(sid * RPT, RPT)],
                    out_hbm.at[cid, pl.ds(sid * RPT, RPT)])

  return agg_kernel(xaug, src3, dst3, zeros)


def _tc_finish_body(agg_ref, x_ref, wl_ref, bl_ref, wr_ref, out_ref):
  a = agg_ref[0] + agg_ref[1]
  deg = a[:, D:D + 1]
  mean = a[:, :D] / jnp.maximum(deg, 1.0)
  out = (
      lax.dot_general(mean, wl_ref[...], (((1,), (1,)), ((), ())),
                      preferred_element_type=jnp.float32)
      + lax.dot_general(x_ref[...], wr_ref[...], (((1,), (1,)), ((), ())),
                        preferred_element_type=jnp.float32)
      + bl_ref[...]
  )
  norm = jnp.sqrt(jnp.sum(out * out, axis=-1, keepdims=True))
  out_ref[...] = out / jnp.maximum(norm, 1e-12)


def _tc_finish(agg2, x, W_l, b_l2, W_r):
  blk = 2000
  grid = N // blk
  return pl.pallas_call(
      _tc_finish_body,
      grid=(grid,),
      in_specs=[
          pl.BlockSpec((NC, blk, DA), lambda i: (0, i, 0)),
          pl.BlockSpec((blk, D), lambda i: (i, 0)),
          pl.BlockSpec((D, D), lambda i: (0, 0)),
          pl.BlockSpec((1, D), lambda i: (0, 0)),
          pl.BlockSpec((D, D), lambda i: (0, 0)),
      ],
      out_specs=pl.BlockSpec((blk, D), lambda i: (i, 0)),
      out_shape=jax.ShapeDtypeStruct((N, D), jnp.float32),
  )(agg2, x, W_l, b_l2, W_r)


@jax.jit
def kernel(x, edge_index, W_l, b_l, W_r):
  xaug = jnp.concatenate(
      [x, jnp.ones((N, 1), jnp.float32), jnp.zeros((N, DA - D - 1), jnp.float32)],
      axis=1)
  src3 = edge_index[0].reshape(NW, NCHUNK, CB)
  dst3 = edge_index[1].reshape(NW, NCHUNK, CB)
  zeros = jnp.zeros((NP, DA), jnp.float32)
  agg2 = _sc_aggregate(xaug, src3, dst3, zeros)
  return _tc_finish(agg2, x, W_l, b_l.reshape(1, D), W_r)
